# baseline (speedup 1.0000x reference)
import jax
import jax.numpy as jnp
from jax import lax
from jax.experimental import pallas as pl
from jax.experimental.pallas import tpu as pltpu
from jax.experimental.pallas import tpu_sc as plsc

NE = 16
T = 8192
D = 4096
BLK = 256

SC_NC = 1
SC_NS = 16
SC_W = SC_NC * SC_NS
TPW = T // SC_W


def _combine_tc_kernel(logits_ref, h_ref, out_ref):
    l = logits_ref[...]
    lane = lax.broadcasted_iota(jnp.int32, l.shape, 1)

    l1 = jnp.max(l, axis=-1, keepdims=True)
    i1 = jnp.min(jnp.where(l == l1, lane, NE), axis=-1, keepdims=True)
    lm = jnp.where(lane == i1, -jnp.inf, l)
    l2 = jnp.max(lm, axis=-1, keepdims=True)

    w1 = 1.0 / (1.0 + jnp.exp(l2 - l1))
    w2 = 1.0 / (1.0 + jnp.exp(l1 - l2))

    h = h_ref[...]
    out_ref[...] = h * w1 + h * w2


UNROLL = 8


def _counts_sc_kernel(logits_hbm, out_hbm, lg_v, sums_v, hist_v, sh_spmem):
    s = lax.axis_index("s")
    pltpu.sync_copy(logits_hbm.at[pl.ds(s * TPW, TPW)], lg_v)
    lane = lax.iota(jnp.int32, NE)
    top2 = lane < 2
    ones = jnp.ones((NE,), jnp.int32)
    hist_v[...] = jnp.zeros((NE,), jnp.int32)

    def body(g, carry):
        for u in range(UNROLL):
            v = lg_v[g * UNROLL + u]
            _, sv = plsc.sort_key_val(v, lane, descending=True)
            plsc.addupdate_scatter(hist_v, [sv], ones, mask=top2)
        return carry

    lax.fori_loop(0, TPW // UNROLL, body, 0)
    pltpu.sync_copy(hist_v, sh_spmem.at[pl.ds(s * NE, NE)])
    plsc.subcore_barrier()

    @pl.when(s == 0)
    def _reduce_core():
        pltpu.sync_copy(sh_spmem, sums_v)
        tot = sums_v[pl.ds(0, NE)]
        for j in range(1, SC_NS):
            tot = tot + sums_v[pl.ds(j * NE, NE)]
        hist_v[...] = tot
        pltpu.sync_copy(hist_v, out_hbm)


_counts_sc = pl.kernel(
    _counts_sc_kernel,
    out_type=jax.ShapeDtypeStruct((NE,), jnp.int32),
    mesh=plsc.VectorSubcoreMesh(core_axis_name="c", subcore_axis_name="s",
                                num_cores=SC_NC, num_subcores=SC_NS),
    scratch_types=[
        pltpu.VMEM((TPW, NE), jnp.float32),
        pltpu.VMEM((SC_NS * NE,), jnp.int32),
        pltpu.VMEM((NE,), jnp.int32),
        pltpu.VMEM_SHARED((SC_NS * NE,), jnp.int32),
    ],
    compiler_params=pltpu.CompilerParams(needs_layout_passes=False),
)


@jax.jit
def kernel(hidden_states, router_logits):
    combined = pl.pallas_call(
        _combine_tc_kernel,
        grid=(T // BLK,),
        in_specs=[
            pl.BlockSpec((BLK, NE), lambda i: (i, 0)),
            pl.BlockSpec((BLK, D), lambda i: (i, 0)),
        ],
        out_specs=pl.BlockSpec((BLK, D), lambda i: (i, 0)),
        out_shape=jax.ShapeDtypeStruct((T, D), jnp.float32),
        compiler_params=pltpu.CompilerParams(
            dimension_semantics=("arbitrary",),
        ),
    )(router_logits, hidden_states)
    counts = _counts_sc(router_logits)
    return combined, counts


# speedup vs baseline: 1.0459x; 1.0459x over previous
import jax
import jax.numpy as jnp
from jax import lax
from jax.experimental import pallas as pl
from jax.experimental.pallas import tpu as pltpu
from jax.experimental.pallas import tpu_sc as plsc

NE = 16
T = 8192
D = 4096
BLK = 512

SC_NC = 1
SC_NS = 16
SC_W = SC_NC * SC_NS
TPW = T // SC_W


def _combine_tc_kernel(logits_ref, h_ref, out_ref):
    l = logits_ref[...]
    lane = lax.broadcasted_iota(jnp.int32, l.shape, 1)

    l1 = jnp.max(l, axis=-1, keepdims=True)
    i1 = jnp.min(jnp.where(l == l1, lane, NE), axis=-1, keepdims=True)
    lm = jnp.where(lane == i1, -jnp.inf, l)
    l2 = jnp.max(lm, axis=-1, keepdims=True)

    w1 = 1.0 / (1.0 + jnp.exp(l2 - l1))
    w2 = 1.0 / (1.0 + jnp.exp(l1 - l2))

    h = h_ref[...]
    out_ref[...] = h * (w1 + w2)


UNROLL = 8


def _counts_sc_kernel(logits_hbm, out_hbm, lg_v, sums_v, hist_v, sh_spmem):
    s = lax.axis_index("s")
    pltpu.sync_copy(logits_hbm.at[pl.ds(s * TPW, TPW)], lg_v)
    lane = lax.iota(jnp.int32, NE)
    top2 = lane < 2
    ones = jnp.ones((NE,), jnp.int32)
    hist_v[...] = jnp.zeros((NE,), jnp.int32)

    def body(g, carry):
        for u in range(UNROLL):
            v = lg_v[g * UNROLL + u]
            _, sv = plsc.sort_key_val(v, lane, descending=True)
            plsc.addupdate_scatter(hist_v, [sv], ones, mask=top2)
        return carry

    lax.fori_loop(0, TPW // UNROLL, body, 0)
    pltpu.sync_copy(hist_v, sh_spmem.at[pl.ds(s * NE, NE)])
    plsc.subcore_barrier()

    @pl.when(s == 0)
    def _reduce_core():
        pltpu.sync_copy(sh_spmem, sums_v)
        tot = sums_v[pl.ds(0, NE)]
        for j in range(1, SC_NS):
            tot = tot + sums_v[pl.ds(j * NE, NE)]
        hist_v[...] = tot
        pltpu.sync_copy(hist_v, out_hbm)


_counts_sc = pl.kernel(
    _counts_sc_kernel,
    out_type=jax.ShapeDtypeStruct((NE,), jnp.int32),
    mesh=plsc.VectorSubcoreMesh(core_axis_name="c", subcore_axis_name="s",
                                num_cores=SC_NC, num_subcores=SC_NS),
    scratch_types=[
        pltpu.VMEM((TPW, NE), jnp.float32),
        pltpu.VMEM((SC_NS * NE,), jnp.int32),
        pltpu.VMEM((NE,), jnp.int32),
        pltpu.VMEM_SHARED((SC_NS * NE,), jnp.int32),
    ],
    compiler_params=pltpu.CompilerParams(needs_layout_passes=False),
)


@jax.jit
def kernel(hidden_states, router_logits):
    combined = pl.pallas_call(
        _combine_tc_kernel,
        grid=(T // BLK,),
        in_specs=[
            pl.BlockSpec((BLK, NE), lambda i: (i, 0)),
            pl.BlockSpec((BLK, D), lambda i: (i, 0)),
        ],
        out_specs=pl.BlockSpec((BLK, D), lambda i: (i, 0)),
        out_shape=jax.ShapeDtypeStruct((T, D), jnp.float32),
        compiler_params=pltpu.CompilerParams(
            dimension_semantics=("arbitrary",),
        ),
    )(router_logits, hidden_states)
    counts = _counts_sc(router_logits)
    return combined, counts
